# baseline (speedup 1.0000x reference)
import functools

import jax
import jax.numpy as jnp
from jax.experimental import pallas as pl
from jax.experimental.pallas import tpu as pltpu

C_SIZE = 2000
T_SIZE = 1000
N = C_SIZE + T_SIZE
CTX = 50
CTXD = 32
GIN = 192
GH = 256
TIME = 100
B = 256
L = 19
TQ = L + 1

F32 = jnp.float32
BF16 = jnp.bfloat16


def _conv_kernel(dur_ref, ppr_ref, int_ref, ext_ref, h_ref,
                 dur_tab_ref, ppr_tab_ref, ctx_w_ref, ctx_b_ref,
                 wi_ref, we_ref, bi_ref, out_ref,
                 acc_int_ref, acc_ext_ref, *, n_blocks):
    i = pl.program_id(0)

    @pl.when(i == 0)
    def _init():
        acc_int_ref[...] = jnp.zeros_like(acc_int_ref)
        acc_ext_ref[...] = jnp.zeros_like(acc_ext_ref)

    durp = jax.lax.dot_general(ctx_w_ref[...], dur_tab_ref[...],
                               (((0,), (1,)), ((), ())),
                               preferred_element_type=F32)
    pprp = jax.lax.dot_general(ctx_w_ref[...], ppr_tab_ref[...],
                               (((0,), (1,)), ((), ())),
                               preferred_element_type=F32)
    rb = dur_ref.shape[0]
    durp_b = jnp.broadcast_to(durp, (rb, CTX))
    pprp_b = jnp.broadcast_to(pprp, (rb, CTX))
    s = (jnp.take_along_axis(durp_b, dur_ref[...], axis=1)
         + jnp.take_along_axis(pprp_b, ppr_ref[...], axis=1)
         + ctx_b_ref[0, 0])
    gate = jax.nn.sigmoid(s).astype(BF16)
    w_edge = gate * int_ref[...].astype(BF16)
    h = h_ref[...].astype(BF16)
    acc_int_ref[...] += jax.lax.dot_general(
        w_edge, h, (((0,), (0,)), ((), ())), preferred_element_type=F32)
    acc_ext_ref[...] += jax.lax.dot_general(
        ext_ref[...].astype(BF16), h, (((0,), (0,)), ((), ())),
        preferred_element_type=F32)

    @pl.when(i == n_blocks - 1)
    def _final():
        h2 = (jnp.dot(acc_int_ref[...], wi_ref[...], preferred_element_type=F32)
              + jnp.dot(acc_ext_ref[...], we_ref[...], preferred_element_type=F32)
              + bi_ref[...])
        out_ref[...] = jnp.maximum(h2, 0.0)


def _conv(dur_g, ppr_g, int_g, ext_g, ct_table, dur_tab, ppr_tab,
          ctx_w, ctx_b, W_int, W_ext, b_int):
    rb = 200
    nb = N // rb
    return pl.pallas_call(
        functools.partial(_conv_kernel, n_blocks=nb),
        grid=(nb,),
        in_specs=[
            pl.BlockSpec((rb, N), lambda i: (i, 0)),
            pl.BlockSpec((rb, N), lambda i: (i, 0)),
            pl.BlockSpec((rb, N), lambda i: (i, 0)),
            pl.BlockSpec((rb, N), lambda i: (i, 0)),
            pl.BlockSpec((rb, GIN), lambda i: (i, 0)),
            pl.BlockSpec((CTX, CTXD), lambda i: (0, 0)),
            pl.BlockSpec((CTX, CTXD), lambda i: (0, 0)),
            pl.BlockSpec((CTXD, 1), lambda i: (0, 0)),
            pl.BlockSpec((1, 1), lambda i: (0, 0)),
            pl.BlockSpec((GIN, GIN), lambda i: (0, 0)),
            pl.BlockSpec((GIN, GIN), lambda i: (0, 0)),
            pl.BlockSpec((1, GIN), lambda i: (0, 0)),
        ],
        out_specs=pl.BlockSpec((N, GIN), lambda i: (0, 0)),
        out_shape=jax.ShapeDtypeStruct((N, GIN), F32),
        scratch_shapes=[pltpu.VMEM((N, GIN), F32), pltpu.VMEM((N, GIN), F32)],
        compiler_params=pltpu.CompilerParams(
            dimension_semantics=("arbitrary",)),
    )(dur_g, ppr_g, int_g, ext_g, ct_table, dur_tab, ppr_tab,
      ctx_w, ctx_b, W_int, W_ext, b_int)


CP = 2048
TP = 1024
TMP = 128
SKP = 384


def _gru_kernel(seqc_ref, seqt_ref, seqtime_ref,
                cemb_ref, temb_ref, ttab_ref,
                sk_ref, skw_ref, skb_ref,
                cx_ref, ch_ref, cb_ref,
                tx_ref, th_ref, tb_ref,
                outc_ref, outt_ref):
    sk = jnp.dot(sk_ref[...], skw_ref[...], preferred_element_type=F32) + skb_ref[...]
    gx_sk_c = jnp.dot(sk, cx_ref[2 * GIN:3 * GIN, :], preferred_element_type=F32)
    gx_sk_t = jnp.dot(sk, tx_ref[2 * GIN:3 * GIN, :], preferred_element_type=F32)

    iota_c = jax.lax.broadcasted_iota(jnp.int32, (1, CP), 1)
    iota_t = jax.lax.broadcasted_iota(jnp.int32, (1, TP), 1)
    iota_m = jax.lax.broadcasted_iota(jnp.int32, (1, TMP), 1)

    hc = jnp.zeros((B, GH), F32)
    ht = jnp.zeros((B, GH), F32)
    for t in range(TQ):
        idc = seqc_ref[:, t:t + 1]
        idt = seqt_ref[:, t:t + 1]
        oh_c = jnp.where(idc == iota_c, 1.0, 0.0)
        oh_t = jnp.where(idt == iota_t, 1.0, 0.0)
        c_t = jnp.dot(oh_c, cemb_ref[...], preferred_element_type=F32)
        t_t = jnp.dot(oh_t, temb_ref[...], preferred_element_type=F32)
        if t == 0:
            te = jnp.zeros((B, GIN), F32)
        else:
            idm = seqtime_ref[:, t - 1:t]
            oh_m = jnp.where(idm == iota_m, 1.0, 0.0)
            te = jnp.dot(oh_m, ttab_ref[...], preferred_element_type=F32)

        gx_c = (jnp.dot(c_t, cx_ref[0:GIN, :], preferred_element_type=F32)
                + jnp.dot(te, cx_ref[GIN:2 * GIN, :], preferred_element_type=F32)
                + gx_sk_c + cb_ref[...])
        gx_t = (jnp.dot(t_t, tx_ref[0:GIN, :], preferred_element_type=F32)
                + jnp.dot(te, tx_ref[GIN:2 * GIN, :], preferred_element_type=F32)
                + gx_sk_t + tb_ref[...])
        gh_c = jnp.dot(hc, ch_ref[...], preferred_element_type=F32)
        gh_t = jnp.dot(ht, th_ref[...], preferred_element_type=F32)

        z = jax.nn.sigmoid(gx_c[:, 0:GH] + gh_c[:, 0:GH])
        r = jax.nn.sigmoid(gx_c[:, GH:2 * GH] + gh_c[:, GH:2 * GH])
        n = jnp.tanh(gx_c[:, 2 * GH:] + r * gh_c[:, 2 * GH:])
        hc = (1.0 - z) * n + z * hc

        z = jax.nn.sigmoid(gx_t[:, 0:GH] + gh_t[:, 0:GH])
        r = jax.nn.sigmoid(gx_t[:, GH:2 * GH] + gh_t[:, GH:2 * GH])
        n = jnp.tanh(gx_t[:, 2 * GH:] + r * gh_t[:, 2 * GH:])
        ht = (1.0 - z) * n + z * ht

        outc_ref[t] = hc
        outt_ref[t] = ht


def _gru(seq_company, seq_title, seq_time, cemb, temb, ttab,
         sk, skw, skb, cx, ch, cb, tx, th, tb):
    return pl.pallas_call(
        _gru_kernel,
        out_shape=[
            jax.ShapeDtypeStruct((TQ, B, GH), F32),
            jax.ShapeDtypeStruct((TQ, B, GH), F32),
        ],
    )(seq_company, seq_title, seq_time, cemb, temb, ttab,
      sk, skw, skb, cx, ch, cb, tx, th, tb)


BB = 64


def _readout_kernel(ch_ref, th_ref, crw_ref, crb_ref, trw_ref, trb_ref,
                    d1w_ref, d1b_ref, d2w_ref, d2b_ref, ac_ref, at_ref,
                    co_ref, to_ref, dur_ref, ca_ref, ta_ref):
    crw = crw_ref[...].astype(BF16)
    trw = trw_ref[...].astype(BF16)
    cl_cols = []
    tl_cols = []
    for t in range(TQ):
        ch = ch_ref[t].astype(BF16)
        th = th_ref[t].astype(BF16)
        co_ref[:, t, :] = (jnp.dot(ch, crw, preferred_element_type=F32)
                           + crb_ref[...])
        to_ref[:, t, :] = (jnp.dot(th, trw, preferred_element_type=F32)
                           + trb_ref[...])
        dcat = jnp.concatenate([ch_ref[t], th_ref[t]], axis=1)
        d1 = jnp.maximum(jnp.dot(dcat, d1w_ref[...], preferred_element_type=F32)
                         + d1b_ref[...], 0.0)
        d2 = jnp.dot(d1, d2w_ref[...], preferred_element_type=F32) + d2b_ref[...]
        dur_ref[:, t:t + 1] = jnp.maximum(d2, 0.0)
        cl_cols.append(jnp.dot(ch_ref[t], ac_ref[...], preferred_element_type=F32))
        tl_cols.append(jnp.dot(th_ref[t], at_ref[...], preferred_element_type=F32))
    for cols, dst in ((cl_cols, ca_ref), (tl_cols, ta_ref)):
        x = jnp.concatenate(cols, axis=1)
        m = jnp.max(x, axis=1, keepdims=True)
        e = jnp.exp(x - m)
        dst[...] = e / jnp.sum(e, axis=1, keepdims=True)


def _readout(ch, th, crw, crb, trw, trb, d1w, d1b, d2w, d2b, ac, at):
    nb = B // BB
    return pl.pallas_call(
        _readout_kernel,
        grid=(nb,),
        in_specs=[
            pl.BlockSpec((TQ, BB, GH), lambda i: (0, i, 0)),
            pl.BlockSpec((TQ, BB, GH), lambda i: (0, i, 0)),
            pl.BlockSpec((GH, C_SIZE), lambda i: (0, 0)),
            pl.BlockSpec((1, C_SIZE), lambda i: (0, 0)),
            pl.BlockSpec((GH, T_SIZE), lambda i: (0, 0)),
            pl.BlockSpec((1, T_SIZE), lambda i: (0, 0)),
            pl.BlockSpec((2 * GH, GH), lambda i: (0, 0)),
            pl.BlockSpec((1, GH), lambda i: (0, 0)),
            pl.BlockSpec((GH, 1), lambda i: (0, 0)),
            pl.BlockSpec((1, 1), lambda i: (0, 0)),
            pl.BlockSpec((GH, 1), lambda i: (0, 0)),
            pl.BlockSpec((GH, 1), lambda i: (0, 0)),
        ],
        out_specs=[
            pl.BlockSpec((BB, TQ, C_SIZE), lambda i: (i, 0, 0)),
            pl.BlockSpec((BB, TQ, T_SIZE), lambda i: (i, 0, 0)),
            pl.BlockSpec((BB, TQ), lambda i: (i, 0)),
            pl.BlockSpec((BB, TQ), lambda i: (i, 0)),
            pl.BlockSpec((BB, TQ), lambda i: (i, 0)),
        ],
        out_shape=[
            jax.ShapeDtypeStruct((B, TQ, C_SIZE), F32),
            jax.ShapeDtypeStruct((B, TQ, T_SIZE), F32),
            jax.ShapeDtypeStruct((B, TQ), F32),
            jax.ShapeDtypeStruct((B, TQ), F32),
            jax.ShapeDtypeStruct((B, TQ), F32),
        ],
        compiler_params=pltpu.CompilerParams(
            dimension_semantics=("arbitrary",)),
    )(ch, th, crw, crb, trw, trb, d1w, d1b, d2w, d2b, ac, at)


def _softmax_kernel(cl_ref, tl_ref, ca_ref, ta_ref):
    for src, dst in ((cl_ref, ca_ref), (tl_ref, ta_ref)):
        x = src[...]
        m = jnp.max(x, axis=1, keepdims=True)
        e = jnp.exp(x - m)
        dst[...] = e / jnp.sum(e, axis=1, keepdims=True)


def _softmax(cl, tl):
    return pl.pallas_call(
        _softmax_kernel,
        out_shape=[jax.ShapeDtypeStruct((B, TQ), F32),
                   jax.ShapeDtypeStruct((B, TQ), F32)],
    )(cl, tl)


def _rebuild_kernel(e_blk_ref, e_ref, w_ref, b_ref, out_ref):
    s = jax.lax.dot_general(e_blk_ref[...].astype(BF16), e_ref[...].astype(BF16),
                            (((1,), (1,)), ((), ())),
                            preferred_element_type=F32)
    out_ref[...] = jax.nn.sigmoid(
        jnp.dot(s.astype(BF16), w_ref[...].astype(BF16),
                preferred_element_type=F32) + b_ref[...])


def _rebuild(e, w, b, size, rb):
    nb = size // rb
    return pl.pallas_call(
        _rebuild_kernel,
        grid=(nb,),
        in_specs=[
            pl.BlockSpec((rb, GIN), lambda i: (i, 0)),
            pl.BlockSpec((size, GIN), lambda i: (0, 0)),
            pl.BlockSpec((size, size), lambda i: (0, 0)),
            pl.BlockSpec((1, size), lambda i: (0, 0)),
        ],
        out_specs=pl.BlockSpec((rb, size), lambda i: (i, 0)),
        out_shape=jax.ShapeDtypeStruct((size, size), F32),
        compiler_params=pltpu.CompilerParams(
            dimension_semantics=("arbitrary",)),
    )(e, e, w, b)


def kernel(seq_company, seq_title, seq_time, dur_context_graph, ppr_context_graph,
           internal_graph, external_graph, batch_skill_embed, dur_ctx_table,
           ppr_ctx_table, ct_table, time_table, skill_w, skill_b, ctx_w, ctx_b,
           W_int, W_ext, b_int, gru_cx, gru_ch, gru_cb, gru_tx, gru_th, gru_tb,
           attn_c, attn_t, cr_w, cr_b, tr_w, tr_b, d1_w, d1_b, d2_w, d2_b,
           cg_w, cg_b, tg_w, tg_b):
    h2 = _conv(dur_context_graph, ppr_context_graph, internal_graph,
               external_graph, ct_table, dur_ctx_table, ppr_ctx_table,
               ctx_w, ctx_b.reshape(1, 1), W_int, W_ext, b_int.reshape(1, GIN))

    cemb = jnp.pad(h2[:C_SIZE], ((0, CP - C_SIZE), (0, 0)))
    temb = jnp.pad(h2[C_SIZE:], ((0, TP - T_SIZE), (0, 0)))
    ttab = jnp.pad(time_table, ((0, TMP - TIME), (0, 0)))
    skp = jnp.pad(batch_skill_embed, ((0, 0), (0, SKP - 300)))
    skwp = jnp.pad(skill_w, ((0, SKP - 300), (0, 0)))

    ch, th = _gru(seq_company.astype(jnp.int32), seq_title.astype(jnp.int32),
                  seq_time.astype(jnp.int32), cemb, temb, ttab,
                  skp, skwp, skill_b.reshape(1, GIN),
                  gru_cx, gru_ch, gru_cb.reshape(1, 3 * GH),
                  gru_tx, gru_th, gru_tb.reshape(1, 3 * GH))

    company_out, title_out, dur, c_attn, t_attn = _readout(
        ch, th, cr_w, cr_b.reshape(1, C_SIZE), tr_w, tr_b.reshape(1, T_SIZE),
        d1_w, d1_b.reshape(1, GH), d2_w, d2_b.reshape(1, 1),
        attn_c.reshape(GH, 1), attn_t.reshape(GH, 1))

    cg = _rebuild(ct_table[:C_SIZE], cg_w, cg_b.reshape(1, C_SIZE), C_SIZE, 400)
    tg = _rebuild(ct_table[C_SIZE:], tg_w, tg_b.reshape(1, T_SIZE), T_SIZE, 200)

    return (company_out, title_out, dur[:, :, None], c_attn, t_attn, cg, tg)


# speedup vs baseline: 1.0254x; 1.0254x over previous
import functools

import jax
import jax.numpy as jnp
from jax.experimental import pallas as pl
from jax.experimental.pallas import tpu as pltpu

C_SIZE = 2000
T_SIZE = 1000
N = C_SIZE + T_SIZE
CTX = 50
CTXD = 32
GIN = 192
GH = 256
TIME = 100
B = 256
L = 19
TQ = L + 1

F32 = jnp.float32
BF16 = jnp.bfloat16


def _conv_kernel(dur_ref, ppr_ref, int_ref, ext_ref, h_ref,
                 dur_tab_ref, ppr_tab_ref, ctx_w_ref, ctx_b_ref,
                 wi_ref, we_ref, bi_ref, out_ref,
                 acc_int_ref, acc_ext_ref, *, n_blocks):
    i = pl.program_id(0)

    @pl.when(i == 0)
    def _init():
        acc_int_ref[...] = jnp.zeros_like(acc_int_ref)
        acc_ext_ref[...] = jnp.zeros_like(acc_ext_ref)

    durp = jax.lax.dot_general(ctx_w_ref[...], dur_tab_ref[...],
                               (((0,), (1,)), ((), ())),
                               preferred_element_type=F32)
    pprp = jax.lax.dot_general(ctx_w_ref[...], ppr_tab_ref[...],
                               (((0,), (1,)), ((), ())),
                               preferred_element_type=F32)
    rb = dur_ref.shape[0]
    durp_b = jnp.broadcast_to(durp, (rb, CTX))
    pprp_b = jnp.broadcast_to(pprp, (rb, CTX))
    s = (jnp.take_along_axis(durp_b, dur_ref[...], axis=1)
         + jnp.take_along_axis(pprp_b, ppr_ref[...], axis=1)
         + ctx_b_ref[0, 0])
    gate = jax.nn.sigmoid(s).astype(BF16)
    w_edge = gate * int_ref[...].astype(BF16)
    h = h_ref[...].astype(BF16)
    acc_int_ref[...] += jax.lax.dot_general(
        h, w_edge, (((0,), (0,)), ((), ())), preferred_element_type=F32)
    acc_ext_ref[...] += jax.lax.dot_general(
        h, ext_ref[...].astype(BF16), (((0,), (0,)), ((), ())),
        preferred_element_type=F32)

    @pl.when(i == n_blocks - 1)
    def _final():
        h2t = (jax.lax.dot_general(wi_ref[...], acc_int_ref[...],
                                   (((0,), (0,)), ((), ())),
                                   preferred_element_type=F32)
               + jax.lax.dot_general(we_ref[...], acc_ext_ref[...],
                                     (((0,), (0,)), ((), ())),
                                     preferred_element_type=F32)
               + bi_ref[...])
        out_ref[...] = jnp.swapaxes(jnp.maximum(h2t, 0.0), 0, 1)


def _conv(dur_g, ppr_g, int_g, ext_g, ct_table, dur_tab, ppr_tab,
          ctx_w, ctx_b, W_int, W_ext, b_int):
    rb = 200
    nb = N // rb
    return pl.pallas_call(
        functools.partial(_conv_kernel, n_blocks=nb),
        grid=(nb,),
        in_specs=[
            pl.BlockSpec((rb, N), lambda i: (i, 0)),
            pl.BlockSpec((rb, N), lambda i: (i, 0)),
            pl.BlockSpec((rb, N), lambda i: (i, 0)),
            pl.BlockSpec((rb, N), lambda i: (i, 0)),
            pl.BlockSpec((rb, GIN), lambda i: (i, 0)),
            pl.BlockSpec((CTX, CTXD), lambda i: (0, 0)),
            pl.BlockSpec((CTX, CTXD), lambda i: (0, 0)),
            pl.BlockSpec((CTXD, 1), lambda i: (0, 0)),
            pl.BlockSpec((1, 1), lambda i: (0, 0)),
            pl.BlockSpec((GIN, GIN), lambda i: (0, 0)),
            pl.BlockSpec((GIN, GIN), lambda i: (0, 0)),
            pl.BlockSpec((GIN, 1), lambda i: (0, 0)),
        ],
        out_specs=pl.BlockSpec((N, GIN), lambda i: (0, 0)),
        out_shape=jax.ShapeDtypeStruct((N, GIN), F32),
        scratch_shapes=[pltpu.VMEM((GIN, N), F32), pltpu.VMEM((GIN, N), F32)],
        compiler_params=pltpu.CompilerParams(
            dimension_semantics=("arbitrary",)),
    )(dur_g, ppr_g, int_g, ext_g, ct_table, dur_tab, ppr_tab,
      ctx_w, ctx_b, W_int, W_ext, b_int)


CP = 2048
TP = 1024
TMP = 128
SKP = 384


def _gru_kernel(seqc_ref, seqt_ref, seqtime_ref,
                cemb_ref, temb_ref, ttab_ref,
                sk_ref, skw_ref, skb_ref,
                cx_ref, ch_ref, cb_ref,
                tx_ref, th_ref, tb_ref,
                outc_ref, outt_ref):
    sk = jnp.dot(sk_ref[...], skw_ref[...], preferred_element_type=F32) + skb_ref[...]
    gx_sk_c = jnp.dot(sk, cx_ref[2 * GIN:3 * GIN, :], preferred_element_type=F32)
    gx_sk_t = jnp.dot(sk, tx_ref[2 * GIN:3 * GIN, :], preferred_element_type=F32)

    iota_c = jax.lax.broadcasted_iota(jnp.int32, (1, CP), 1)
    iota_t = jax.lax.broadcasted_iota(jnp.int32, (1, TP), 1)
    iota_m = jax.lax.broadcasted_iota(jnp.int32, (1, TMP), 1)

    hc = jnp.zeros((B, GH), F32)
    ht = jnp.zeros((B, GH), F32)
    for t in range(TQ):
        idc = seqc_ref[:, t:t + 1]
        idt = seqt_ref[:, t:t + 1]
        oh_c = jnp.where(idc == iota_c, 1.0, 0.0)
        oh_t = jnp.where(idt == iota_t, 1.0, 0.0)
        c_t = jnp.dot(oh_c, cemb_ref[...], preferred_element_type=F32)
        t_t = jnp.dot(oh_t, temb_ref[...], preferred_element_type=F32)
        if t == 0:
            te = jnp.zeros((B, GIN), F32)
        else:
            idm = seqtime_ref[:, t - 1:t]
            oh_m = jnp.where(idm == iota_m, 1.0, 0.0)
            te = jnp.dot(oh_m, ttab_ref[...], preferred_element_type=F32)

        gx_c = (jnp.dot(c_t, cx_ref[0:GIN, :], preferred_element_type=F32)
                + jnp.dot(te, cx_ref[GIN:2 * GIN, :], preferred_element_type=F32)
                + gx_sk_c + cb_ref[...])
        gx_t = (jnp.dot(t_t, tx_ref[0:GIN, :], preferred_element_type=F32)
                + jnp.dot(te, tx_ref[GIN:2 * GIN, :], preferred_element_type=F32)
                + gx_sk_t + tb_ref[...])
        gh_c = jnp.dot(hc, ch_ref[...], preferred_element_type=F32)
        gh_t = jnp.dot(ht, th_ref[...], preferred_element_type=F32)

        z = jax.nn.sigmoid(gx_c[:, 0:GH] + gh_c[:, 0:GH])
        r = jax.nn.sigmoid(gx_c[:, GH:2 * GH] + gh_c[:, GH:2 * GH])
        n = jnp.tanh(gx_c[:, 2 * GH:] + r * gh_c[:, 2 * GH:])
        hc = (1.0 - z) * n + z * hc

        z = jax.nn.sigmoid(gx_t[:, 0:GH] + gh_t[:, 0:GH])
        r = jax.nn.sigmoid(gx_t[:, GH:2 * GH] + gh_t[:, GH:2 * GH])
        n = jnp.tanh(gx_t[:, 2 * GH:] + r * gh_t[:, 2 * GH:])
        ht = (1.0 - z) * n + z * ht

        outc_ref[t] = hc
        outt_ref[t] = ht


def _gru(seq_company, seq_title, seq_time, cemb, temb, ttab,
         sk, skw, skb, cx, ch, cb, tx, th, tb):
    return pl.pallas_call(
        _gru_kernel,
        out_shape=[
            jax.ShapeDtypeStruct((TQ, B, GH), F32),
            jax.ShapeDtypeStruct((TQ, B, GH), F32),
        ],
    )(seq_company, seq_title, seq_time, cemb, temb, ttab,
      sk, skw, skb, cx, ch, cb, tx, th, tb)


BB = 64


def _readout_kernel(ch_ref, th_ref, crw_ref, crb_ref, trw_ref, trb_ref,
                    d1w_ref, d1b_ref, d2w_ref, d2b_ref, ac_ref, at_ref,
                    co_ref, to_ref, dur_ref, ca_ref, ta_ref):
    chb = jnp.swapaxes(ch_ref[...], 0, 1)
    thb = jnp.swapaxes(th_ref[...], 0, 1)
    ch2 = chb.reshape(BB * TQ, GH)
    th2 = thb.reshape(BB * TQ, GH)
    co_ref[...] = (jnp.dot(ch2.astype(BF16), crw_ref[...].astype(BF16),
                           preferred_element_type=F32) + crb_ref[...])
    to_ref[...] = (jnp.dot(th2.astype(BF16), trw_ref[...].astype(BF16),
                           preferred_element_type=F32) + trb_ref[...])
    dcat = jnp.concatenate([ch2, th2], axis=1)
    d1 = jnp.maximum(jnp.dot(dcat, d1w_ref[...], preferred_element_type=F32)
                     + d1b_ref[...], 0.0)
    d13 = d1.reshape(BB, TQ, GH)
    dur_ref[...] = jnp.maximum(
        jnp.sum(d13 * d2w_ref[...].reshape(1, 1, GH), axis=2) + d2b_ref[0, 0], 0.0)
    for hb, vec_ref, dst in ((chb, ac_ref, ca_ref), (thb, at_ref, ta_ref)):
        x = jnp.sum(hb * vec_ref[...].reshape(1, 1, GH), axis=2)
        m = jnp.max(x, axis=1, keepdims=True)
        e = jnp.exp(x - m)
        dst[...] = e / jnp.sum(e, axis=1, keepdims=True)


def _readout(ch, th, crw, crb, trw, trb, d1w, d1b, d2w, d2b, ac, at):
    nb = B // BB
    return pl.pallas_call(
        _readout_kernel,
        grid=(nb,),
        in_specs=[
            pl.BlockSpec((TQ, BB, GH), lambda i: (0, i, 0)),
            pl.BlockSpec((TQ, BB, GH), lambda i: (0, i, 0)),
            pl.BlockSpec((GH, C_SIZE), lambda i: (0, 0)),
            pl.BlockSpec((1, C_SIZE), lambda i: (0, 0)),
            pl.BlockSpec((GH, T_SIZE), lambda i: (0, 0)),
            pl.BlockSpec((1, T_SIZE), lambda i: (0, 0)),
            pl.BlockSpec((2 * GH, GH), lambda i: (0, 0)),
            pl.BlockSpec((1, GH), lambda i: (0, 0)),
            pl.BlockSpec((1, GH), lambda i: (0, 0)),
            pl.BlockSpec((1, 1), lambda i: (0, 0)),
            pl.BlockSpec((1, GH), lambda i: (0, 0)),
            pl.BlockSpec((1, GH), lambda i: (0, 0)),
        ],
        out_specs=[
            pl.BlockSpec((BB * TQ, C_SIZE), lambda i: (i, 0)),
            pl.BlockSpec((BB * TQ, T_SIZE), lambda i: (i, 0)),
            pl.BlockSpec((BB, TQ), lambda i: (i, 0)),
            pl.BlockSpec((BB, TQ), lambda i: (i, 0)),
            pl.BlockSpec((BB, TQ), lambda i: (i, 0)),
        ],
        out_shape=[
            jax.ShapeDtypeStruct((B * TQ, C_SIZE), F32),
            jax.ShapeDtypeStruct((B * TQ, T_SIZE), F32),
            jax.ShapeDtypeStruct((B, TQ), F32),
            jax.ShapeDtypeStruct((B, TQ), F32),
            jax.ShapeDtypeStruct((B, TQ), F32),
        ],
        compiler_params=pltpu.CompilerParams(
            dimension_semantics=("arbitrary",)),
    )(ch, th, crw, crb, trw, trb, d1w, d1b, d2w, d2b, ac, at)


def _softmax_kernel(cl_ref, tl_ref, ca_ref, ta_ref):
    for src, dst in ((cl_ref, ca_ref), (tl_ref, ta_ref)):
        x = src[...]
        m = jnp.max(x, axis=1, keepdims=True)
        e = jnp.exp(x - m)
        dst[...] = e / jnp.sum(e, axis=1, keepdims=True)


def _softmax(cl, tl):
    return pl.pallas_call(
        _softmax_kernel,
        out_shape=[jax.ShapeDtypeStruct((B, TQ), F32),
                   jax.ShapeDtypeStruct((B, TQ), F32)],
    )(cl, tl)


def _rebuild_kernel(e_blk_ref, e_ref, w_ref, b_ref, out_ref):
    s = jax.lax.dot_general(e_blk_ref[...].astype(BF16), e_ref[...].astype(BF16),
                            (((1,), (1,)), ((), ())),
                            preferred_element_type=F32)
    out_ref[...] = jax.nn.sigmoid(
        jnp.dot(s.astype(BF16), w_ref[...].astype(BF16),
                preferred_element_type=F32) + b_ref[...])


def _rebuild(e, w, b, size, rb):
    nb = size // rb
    return pl.pallas_call(
        _rebuild_kernel,
        grid=(nb,),
        in_specs=[
            pl.BlockSpec((rb, GIN), lambda i: (i, 0)),
            pl.BlockSpec((size, GIN), lambda i: (0, 0)),
            pl.BlockSpec((size, size), lambda i: (0, 0)),
            pl.BlockSpec((1, size), lambda i: (0, 0)),
        ],
        out_specs=pl.BlockSpec((rb, size), lambda i: (i, 0)),
        out_shape=jax.ShapeDtypeStruct((size, size), F32),
        compiler_params=pltpu.CompilerParams(
            dimension_semantics=("arbitrary",)),
    )(e, e, w, b)


def kernel(seq_company, seq_title, seq_time, dur_context_graph, ppr_context_graph,
           internal_graph, external_graph, batch_skill_embed, dur_ctx_table,
           ppr_ctx_table, ct_table, time_table, skill_w, skill_b, ctx_w, ctx_b,
           W_int, W_ext, b_int, gru_cx, gru_ch, gru_cb, gru_tx, gru_th, gru_tb,
           attn_c, attn_t, cr_w, cr_b, tr_w, tr_b, d1_w, d1_b, d2_w, d2_b,
           cg_w, cg_b, tg_w, tg_b):
    h2 = _conv(dur_context_graph, ppr_context_graph, internal_graph,
               external_graph, ct_table, dur_ctx_table, ppr_ctx_table,
               ctx_w, ctx_b.reshape(1, 1), W_int, W_ext, b_int.reshape(GIN, 1))

    cemb = jnp.pad(h2[:C_SIZE], ((0, CP - C_SIZE), (0, 0)))
    temb = jnp.pad(h2[C_SIZE:], ((0, TP - T_SIZE), (0, 0)))
    ttab = jnp.pad(time_table, ((0, TMP - TIME), (0, 0)))
    skp = jnp.pad(batch_skill_embed, ((0, 0), (0, SKP - 300)))
    skwp = jnp.pad(skill_w, ((0, SKP - 300), (0, 0)))

    ch, th = _gru(seq_company.astype(jnp.int32), seq_title.astype(jnp.int32),
                  seq_time.astype(jnp.int32), cemb, temb, ttab,
                  skp, skwp, skill_b.reshape(1, GIN),
                  gru_cx, gru_ch, gru_cb.reshape(1, 3 * GH),
                  gru_tx, gru_th, gru_tb.reshape(1, 3 * GH))

    co, to, dur, c_attn, t_attn = _readout(
        ch, th, cr_w, cr_b.reshape(1, C_SIZE), tr_w, tr_b.reshape(1, T_SIZE),
        d1_w, d1_b.reshape(1, GH), d2_w.reshape(1, GH), d2_b.reshape(1, 1),
        attn_c.reshape(1, GH), attn_t.reshape(1, GH))
    company_out = co.reshape(B, TQ, C_SIZE)
    title_out = to.reshape(B, TQ, T_SIZE)

    cg = _rebuild(ct_table[:C_SIZE], cg_w, cg_b.reshape(1, C_SIZE), C_SIZE, 400)
    tg = _rebuild(ct_table[C_SIZE:], tg_w, tg_b.reshape(1, T_SIZE), T_SIZE, 200)

    return (company_out, title_out, dur[:, :, None], c_attn, t_attn, cg, tg)
